# Initial kernel scaffold; baseline (speedup 1.0000x reference)
import functools

import jax
import jax.numpy as jnp
from jax import lax
from jax.experimental import pallas as pl
from jax.experimental.pallas import tpu as pltpu
from jax.experimental.pallas import tpu_sc as plsc

_NC = 2
_NS = 16
_NW = _NC * _NS

_MESH = plsc.VectorSubcoreMesh(
    core_axis_name="c", subcore_axis_name="s", num_cores=_NC, num_subcores=_NS
)

_CH = 128


def _zero_vmem(buf, n_rows, width):
    z16 = jnp.zeros((16,), jnp.float32)

    def body(i, c):
        for j in range(width // 16):
            buf[i, pl.ds(16 * j, 16)] = z16
        return c

    lax.fori_loop(0, n_rows, body, 0)


def _make_degree(N, E):
    EW = E // _NW
    n_full, tail = divmod(EW, _CH)
    assert EW * _NW == E and tail % 8 == 0
    ROWS_W = N // _NS
    ZR = 125
    assert ROWS_W % ZR == 0

    @functools.partial(
        pl.kernel,
        out_type=jax.ShapeDtypeStruct((_NC, N, 16), jnp.float32),
        mesh=_MESH,
        scratch_types=[
            pltpu.VMEM((_CH,), jnp.int32),
            pltpu.VMEM((16,), jnp.int32),
            pltpu.VMEM((_CH, 16), jnp.float32),
            pltpu.VMEM((ZR, 16), jnp.float32),
            pltpu.VMEM_SHARED((N, 16), jnp.float32),
        ],
    )
    def deg_kernel(dst_hbm, out_hbm, didx, didx_t, ones, zbuf, acc):
        cid = lax.axis_index("c")
        sid = lax.axis_index("s")
        one16 = jnp.ones((16,), jnp.float32)

        def ones_body(i, c):
            ones[i, pl.ds(0, 16)] = one16
            return c

        lax.fori_loop(0, _CH, ones_body, 0)
        _zero_vmem(zbuf, ZR, 16)
        row0 = sid * ROWS_W

        def zero_body(k, c):
            pltpu.sync_copy(zbuf, acc.at[pl.ds(row0 + k * ZR, ZR)])
            return c

        lax.fori_loop(0, ROWS_W // ZR, zero_body, 0)
        plsc.subcore_barrier()

        base_w = cid * (E // _NC) + sid * EW

        def chunk(k, c):
            base = pl.multiple_of(base_w + k * _CH, 8)
            pltpu.sync_copy(dst_hbm.at[pl.ds(base, _CH)], didx)
            pltpu.sync_copy(ones, acc.at[didx], add=True)
            return c

        lax.fori_loop(0, n_full, chunk, 0)
        if tail:
            base = pl.multiple_of(base_w + n_full * _CH, 8)
            pltpu.sync_copy(dst_hbm.at[pl.ds(base, tail)], didx_t)
            pltpu.sync_copy(ones.at[pl.ds(0, tail)], acc.at[didx_t], add=True)
        plsc.subcore_barrier()

        def out_body(k, c):
            r = row0 + k * ZR
            pltpu.sync_copy(acc.at[pl.ds(r, ZR)], zbuf)
            pltpu.sync_copy(zbuf, out_hbm.at[cid, pl.ds(r, ZR)])
            return c

        lax.fori_loop(0, ROWS_W // ZR, out_body, 0)

    return deg_kernel


def _make_prop(N, E, D):
    EW = E // _NW
    n_full, tail = divmod(EW, _CH)
    assert EW * _NW == E and tail % 8 == 0
    ROWS_W = N // _NS
    ZR = 125
    assert ROWS_W % ZR == 0

    @functools.partial(
        pl.kernel,
        out_type=jax.ShapeDtypeStruct((_NC, N, D), jnp.float32),
        mesh=_MESH,
        scratch_types=[
            pltpu.VMEM((_CH,), jnp.int32),
            pltpu.VMEM((_CH,), jnp.int32),
            pltpu.VMEM((16,), jnp.int32),
            pltpu.VMEM((16,), jnp.int32),
            pltpu.VMEM((_CH, D), jnp.float32),
            pltpu.VMEM((16, D), jnp.float32),
            pltpu.VMEM((ZR, D), jnp.float32),
            pltpu.VMEM_SHARED((N, D), jnp.float32),
            pltpu.SemaphoreType.DMA,
        ],
    )
    def prop_kernel(
        u_hbm, src_hbm, dst_hbm, out_hbm,
        sidx, didx, sidx_t, didx_t, rows, rows_t, zbuf, acc, sem,
    ):
        cid = lax.axis_index("c")
        sid = lax.axis_index("s")
        _zero_vmem(zbuf, ZR, D)
        row0 = sid * ROWS_W

        def zero_body(k, c):
            pltpu.sync_copy(zbuf, acc.at[pl.ds(row0 + k * ZR, ZR)])
            return c

        lax.fori_loop(0, ROWS_W // ZR, zero_body, 0)
        plsc.subcore_barrier()

        base_w = cid * (E // _NC) + sid * EW

        def chunk(k, c):
            base = pl.multiple_of(base_w + k * _CH, 8)
            pltpu.sync_copy(src_hbm.at[pl.ds(base, _CH)], sidx)
            pltpu.async_copy(u_hbm.at[sidx], rows, sem).wait()
            pltpu.sync_copy(dst_hbm.at[pl.ds(base, _CH)], didx)
            pltpu.sync_copy(rows, acc.at[didx], add=True)
            return c

        lax.fori_loop(0, n_full, chunk, 0)
        if tail:
            base = pl.multiple_of(base_w + n_full * _CH, 8)
            pltpu.sync_copy(src_hbm.at[pl.ds(base, tail)], sidx_t)
            pltpu.async_copy(u_hbm.at[sidx_t], rows_t, sem).wait()
            pltpu.sync_copy(dst_hbm.at[pl.ds(base, tail)], didx_t)
            pltpu.sync_copy(rows_t, acc.at[didx_t], add=True)
        plsc.subcore_barrier()

        def out_body(k, c):
            r = row0 + k * ZR
            pltpu.sync_copy(acc.at[pl.ds(r, ZR)], zbuf)
            pltpu.sync_copy(zbuf, out_hbm.at[cid, pl.ds(r, ZR)])
            return c

        lax.fori_loop(0, ROWS_W // ZR, out_body, 0)

    return prop_kernel


_BLK = 1000


def _tc_grid(N):
    assert N % _BLK == 0
    return N // _BLK


def _dinv_u_kernel(degp_ref, x_ref, dinv_ref, u_ref):
    deg = degp_ref[0, :, 0:1] + degp_ref[1, :, 0:1] + 1.0
    dv = lax.rsqrt(deg)
    dinv_ref[...] = dv
    u_ref[...] = dv * x_ref[...]


def _dinv_and_u(degp, x):
    N, D = x.shape
    return pl.pallas_call(
        _dinv_u_kernel,
        grid=(_tc_grid(N),),
        in_specs=[
            pl.BlockSpec((_NC, _BLK, 16), lambda i: (0, i, 0)),
            pl.BlockSpec((_BLK, D), lambda i: (i, 0)),
        ],
        out_specs=[
            pl.BlockSpec((_BLK, 1), lambda i: (i, 0)),
            pl.BlockSpec((_BLK, D), lambda i: (i, 0)),
        ],
        out_shape=[
            jax.ShapeDtypeStruct((N, 1), jnp.float32),
            jax.ShapeDtypeStruct((N, D), jnp.float32),
        ],
    )(degp, x)


def _matmul_in_kernel(rp_ref, u_ref, dinv_ref, w_ref, b_ref, o_ref):
    a = dinv_ref[...] * (rp_ref[0] + rp_ref[1] + u_ref[...])
    h = jnp.dot(a, w_ref[...], preferred_element_type=jnp.float32)
    o_ref[...] = jnp.maximum(h + b_ref[...], 0.0)


def _prop_matmul_relu(rp, u, dinv, w, b):
    N, D = u.shape
    K = w.shape[1]
    return pl.pallas_call(
        _matmul_in_kernel,
        grid=(_tc_grid(N),),
        in_specs=[
            pl.BlockSpec((_NC, _BLK, D), lambda i: (0, i, 0)),
            pl.BlockSpec((_BLK, D), lambda i: (i, 0)),
            pl.BlockSpec((_BLK, 1), lambda i: (i, 0)),
            pl.BlockSpec((D, K), lambda i: (0, 0)),
            pl.BlockSpec((1, K), lambda i: (0, 0)),
        ],
        out_specs=pl.BlockSpec((_BLK, K), lambda i: (i, 0)),
        out_shape=jax.ShapeDtypeStruct((N, K), jnp.float32),
    )(rp, u, dinv, w, b)


def _matmul_out_kernel(h_ref, w_ref, dinv_ref, o_ref):
    t = jnp.dot(h_ref[...], w_ref[...], preferred_element_type=jnp.float32)
    o_ref[...] = dinv_ref[...] * t


def _matmul_scale(h, w, dinv):
    N, D = h.shape
    K = w.shape[1]
    return pl.pallas_call(
        _matmul_out_kernel,
        grid=(_tc_grid(N),),
        in_specs=[
            pl.BlockSpec((_BLK, D), lambda i: (i, 0)),
            pl.BlockSpec((D, K), lambda i: (0, 0)),
            pl.BlockSpec((_BLK, 1), lambda i: (i, 0)),
        ],
        out_specs=pl.BlockSpec((_BLK, K), lambda i: (i, 0)),
        out_shape=jax.ShapeDtypeStruct((N, K), jnp.float32),
    )(h, w, dinv)


def _combine2_kernel(rp_ref, u_ref, dinv_ref, b_ref, z_ref, un_ref):
    dv = dinv_ref[...]
    z = dv * (rp_ref[0] + rp_ref[1] + u_ref[...]) + b_ref[...]
    z_ref[...] = z
    un_ref[...] = dv * z


def _combine2(rp, u, dinv, b):
    N, D = u.shape
    return pl.pallas_call(
        _combine2_kernel,
        grid=(_tc_grid(N),),
        in_specs=[
            pl.BlockSpec((_NC, _BLK, D), lambda i: (0, i, 0)),
            pl.BlockSpec((_BLK, D), lambda i: (i, 0)),
            pl.BlockSpec((_BLK, 1), lambda i: (i, 0)),
            pl.BlockSpec((1, D), lambda i: (0, 0)),
        ],
        out_specs=[
            pl.BlockSpec((_BLK, D), lambda i: (i, 0)),
            pl.BlockSpec((_BLK, D), lambda i: (i, 0)),
        ],
        out_shape=[
            jax.ShapeDtypeStruct((N, D), jnp.float32),
            jax.ShapeDtypeStruct((N, D), jnp.float32),
        ],
    )(rp, u, dinv, b)


def _combine1_kernel(rp_ref, u_ref, dinv_ref, b_ref, z_ref):
    z_ref[...] = dinv_ref[...] * (rp_ref[0] + rp_ref[1] + u_ref[...]) + b_ref[...]


def _combine1(rp, u, dinv, b):
    N, D = u.shape
    return pl.pallas_call(
        _combine1_kernel,
        grid=(_tc_grid(N),),
        in_specs=[
            pl.BlockSpec((_NC, _BLK, D), lambda i: (0, i, 0)),
            pl.BlockSpec((_BLK, D), lambda i: (i, 0)),
            pl.BlockSpec((_BLK, 1), lambda i: (i, 0)),
            pl.BlockSpec((1, D), lambda i: (0, 0)),
        ],
        out_specs=pl.BlockSpec((_BLK, D), lambda i: (i, 0)),
        out_shape=jax.ShapeDtypeStruct((N, D), jnp.float32),
    )(rp, u, dinv, b)


def kernel(x, edge_index, W1, b1, W2, b2, W3, b3, W4, b4):
    N, D = x.shape
    E = edge_index.shape[1]
    ei = edge_index.astype(jnp.int32)
    src, dst = ei[0], ei[1]

    degree = _make_degree(N, E)
    prop = _make_prop(N, E, D)

    degp = degree(dst)
    dinv, u1 = _dinv_and_u(degp, x)

    r1 = prop(u1, src, dst)
    h = _prop_matmul_relu(r1, u1, dinv, W1, b1.reshape(1, -1))
    u2 = _matmul_scale(h, W2, dinv)

    r2 = prop(u2, src, dst)
    z, u3 = _combine2(r2, u2, dinv, b2.reshape(1, -1))

    r3 = prop(u3, src, dst)
    h2 = _prop_matmul_relu(r3, u3, dinv, W3, b3.reshape(1, -1))
    u4 = _matmul_scale(h2, W4, dinv)

    r4 = prop(u4, src, dst)
    recon = _combine1(r4, u4, dinv, b4.reshape(1, -1))
    return (z, recon)


# speedup vs baseline: 14.1970x; 14.1970x over previous
import functools

import jax
import jax.numpy as jnp
from jax import lax
from jax.experimental import pallas as pl
from jax.experimental.pallas import tpu as pltpu
from jax.experimental.pallas import tpu_sc as plsc

_NC = 2
_NS = 16
_NW = _NC * _NS

_MESH = plsc.VectorSubcoreMesh(
    core_axis_name="c", subcore_axis_name="s", num_cores=_NC, num_subcores=_NS
)

_CH = 128

_NZW = 10


def _zero_vmem(buf, n_rows, width):
    z16 = jnp.zeros((16,), jnp.float32)

    def body(i, c):
        for j in range(width // 16):
            buf[i, pl.ds(16 * j, 16)] = z16
        return c

    lax.fori_loop(0, n_rows, body, 0)


def _make_degree(N, E):
    EW = E // _NW
    n_full, tail = divmod(EW, _CH)
    assert EW * _NW == E and tail % 8 == 0
    ROWS_Z = N // _NZW
    ZR = ROWS_Z // 5
    assert ROWS_Z * _NZW == N and ZR * 5 == ROWS_Z and ZR % 8 == 0

    @functools.partial(
        pl.kernel,
        out_type=jax.ShapeDtypeStruct((_NC, N, 16), jnp.float32),
        mesh=_MESH,
        scratch_types=[
            pltpu.VMEM((_CH,), jnp.int32),
            pltpu.VMEM((16,), jnp.int32),
            pltpu.VMEM((_CH, 16), jnp.float32),
            pltpu.VMEM((ZR, 16), jnp.float32),
            pltpu.VMEM_SHARED((N, 16), jnp.float32),
        ],
    )
    def deg_kernel(dst_hbm, out_hbm, didx, didx_t, ones, zbuf, acc):
        cid = lax.axis_index("c")
        sid = lax.axis_index("s")
        one16 = jnp.ones((16,), jnp.float32)

        def ones_body(i, c):
            ones[i, pl.ds(0, 16)] = one16
            return c

        lax.fori_loop(0, _CH, ones_body, 0)
        _zero_vmem(zbuf, ZR, 16)
        row0 = sid * ROWS_Z

        @pl.when(sid < _NZW)
        def _():
            def zero_body(k, c):
                pltpu.sync_copy(zbuf, acc.at[pl.ds(row0 + k * ZR, ZR)])
                return c

            lax.fori_loop(0, 5, zero_body, 0)

        plsc.subcore_barrier()

        base_w = cid * (E // _NC) + sid * EW

        def chunk(k, c):
            base = pl.multiple_of(base_w + k * _CH, 8)
            pltpu.sync_copy(dst_hbm.at[pl.ds(base, _CH)], didx)
            pltpu.sync_copy(ones, acc.at[didx], add=True)
            return c

        lax.fori_loop(0, n_full, chunk, 0)
        if tail:
            base = pl.multiple_of(base_w + n_full * _CH, 8)
            pltpu.sync_copy(dst_hbm.at[pl.ds(base, tail)], didx_t)
            pltpu.sync_copy(ones.at[pl.ds(0, tail)], acc.at[didx_t], add=True)
        plsc.subcore_barrier()

        @pl.when(sid < _NZW)
        def _():
            def out_body(k, c):
                r = row0 + k * ZR
                pltpu.sync_copy(acc.at[pl.ds(r, ZR)], zbuf)
                pltpu.sync_copy(zbuf, out_hbm.at[cid, pl.ds(r, ZR)])
                return c

            lax.fori_loop(0, 5, out_body, 0)

    return deg_kernel


def _make_prop(N, E, D):
    EW = E // _NW
    n_full, tail = divmod(EW, _CH)
    assert EW * _NW == E and tail % 8 == 0
    ROWS_Z = N // _NZW
    ZR = ROWS_Z // 5
    assert ROWS_Z * _NZW == N and ZR * 5 == ROWS_Z and ZR % 8 == 0

    @functools.partial(
        pl.kernel,
        out_type=jax.ShapeDtypeStruct((_NC, N, D), jnp.float32),
        mesh=_MESH,
        scratch_types=[
            pltpu.VMEM((_CH,), jnp.int32),
            pltpu.VMEM((_CH,), jnp.int32),
            pltpu.VMEM((16,), jnp.int32),
            pltpu.VMEM((16,), jnp.int32),
            pltpu.VMEM((_CH, D), jnp.float32),
            pltpu.VMEM((16, D), jnp.float32),
            pltpu.VMEM((ZR, D), jnp.float32),
            pltpu.VMEM_SHARED((N, D), jnp.float32),
            pltpu.SemaphoreType.DMA,
        ],
    )
    def prop_kernel(
        u_hbm, src_hbm, dst_hbm, out_hbm,
        sidx, didx, sidx_t, didx_t, rows, rows_t, zbuf, acc, sem,
    ):
        cid = lax.axis_index("c")
        sid = lax.axis_index("s")
        _zero_vmem(zbuf, ZR, D)
        row0 = sid * ROWS_Z

        @pl.when(sid < _NZW)
        def _():
            def zero_body(k, c):
                pltpu.sync_copy(zbuf, acc.at[pl.ds(row0 + k * ZR, ZR)])
                return c

            lax.fori_loop(0, 5, zero_body, 0)

        plsc.subcore_barrier()

        base_w = cid * (E // _NC) + sid * EW

        def chunk(k, c):
            base = pl.multiple_of(base_w + k * _CH, 8)
            pltpu.sync_copy(src_hbm.at[pl.ds(base, _CH)], sidx)
            pltpu.async_copy(u_hbm.at[sidx], rows, sem).wait()
            pltpu.sync_copy(dst_hbm.at[pl.ds(base, _CH)], didx)
            pltpu.sync_copy(rows, acc.at[didx], add=True)
            return c

        lax.fori_loop(0, n_full, chunk, 0)
        if tail:
            base = pl.multiple_of(base_w + n_full * _CH, 8)
            pltpu.sync_copy(src_hbm.at[pl.ds(base, tail)], sidx_t)
            pltpu.async_copy(u_hbm.at[sidx_t], rows_t, sem).wait()
            pltpu.sync_copy(dst_hbm.at[pl.ds(base, tail)], didx_t)
            pltpu.sync_copy(rows_t, acc.at[didx_t], add=True)
        plsc.subcore_barrier()

        @pl.when(sid < _NZW)
        def _():
            def out_body(k, c):
                r = row0 + k * ZR
                pltpu.sync_copy(acc.at[pl.ds(r, ZR)], zbuf)
                pltpu.sync_copy(zbuf, out_hbm.at[cid, pl.ds(r, ZR)])
                return c

            lax.fori_loop(0, 5, out_body, 0)

    return prop_kernel


_BLK = 1000


def _tc_grid(N):
    assert N % _BLK == 0
    return N // _BLK


def _dinv_u_kernel(degp_ref, x_ref, dinv_ref, u_ref):
    deg = degp_ref[0, :, 0:1] + degp_ref[1, :, 0:1] + 1.0
    dv = lax.rsqrt(deg)
    dinv_ref[...] = dv
    u_ref[...] = dv * x_ref[...]


def _dinv_and_u(degp, x):
    N, D = x.shape
    return pl.pallas_call(
        _dinv_u_kernel,
        grid=(_tc_grid(N),),
        in_specs=[
            pl.BlockSpec((_NC, _BLK, 16), lambda i: (0, i, 0)),
            pl.BlockSpec((_BLK, D), lambda i: (i, 0)),
        ],
        out_specs=[
            pl.BlockSpec((_BLK, 1), lambda i: (i, 0)),
            pl.BlockSpec((_BLK, D), lambda i: (i, 0)),
        ],
        out_shape=[
            jax.ShapeDtypeStruct((N, 1), jnp.float32),
            jax.ShapeDtypeStruct((N, D), jnp.float32),
        ],
    )(degp, x)


def _matmul_in_kernel(rp_ref, u_ref, dinv_ref, w_ref, b_ref, o_ref):
    a = dinv_ref[...] * (rp_ref[0] + rp_ref[1] + u_ref[...])
    h = jnp.dot(a, w_ref[...], preferred_element_type=jnp.float32)
    o_ref[...] = jnp.maximum(h + b_ref[...], 0.0)


def _prop_matmul_relu(rp, u, dinv, w, b):
    N, D = u.shape
    K = w.shape[1]
    return pl.pallas_call(
        _matmul_in_kernel,
        grid=(_tc_grid(N),),
        in_specs=[
            pl.BlockSpec((_NC, _BLK, D), lambda i: (0, i, 0)),
            pl.BlockSpec((_BLK, D), lambda i: (i, 0)),
            pl.BlockSpec((_BLK, 1), lambda i: (i, 0)),
            pl.BlockSpec((D, K), lambda i: (0, 0)),
            pl.BlockSpec((1, K), lambda i: (0, 0)),
        ],
        out_specs=pl.BlockSpec((_BLK, K), lambda i: (i, 0)),
        out_shape=jax.ShapeDtypeStruct((N, K), jnp.float32),
    )(rp, u, dinv, w, b)


def _matmul_out_kernel(h_ref, w_ref, dinv_ref, o_ref):
    t = jnp.dot(h_ref[...], w_ref[...], preferred_element_type=jnp.float32)
    o_ref[...] = dinv_ref[...] * t


def _matmul_scale(h, w, dinv):
    N, D = h.shape
    K = w.shape[1]
    return pl.pallas_call(
        _matmul_out_kernel,
        grid=(_tc_grid(N),),
        in_specs=[
            pl.BlockSpec((_BLK, D), lambda i: (i, 0)),
            pl.BlockSpec((D, K), lambda i: (0, 0)),
            pl.BlockSpec((_BLK, 1), lambda i: (i, 0)),
        ],
        out_specs=pl.BlockSpec((_BLK, K), lambda i: (i, 0)),
        out_shape=jax.ShapeDtypeStruct((N, K), jnp.float32),
    )(h, w, dinv)


def _combine2_kernel(rp_ref, u_ref, dinv_ref, b_ref, z_ref, un_ref):
    dv = dinv_ref[...]
    z = dv * (rp_ref[0] + rp_ref[1] + u_ref[...]) + b_ref[...]
    z_ref[...] = z
    un_ref[...] = dv * z


def _combine2(rp, u, dinv, b):
    N, D = u.shape
    return pl.pallas_call(
        _combine2_kernel,
        grid=(_tc_grid(N),),
        in_specs=[
            pl.BlockSpec((_NC, _BLK, D), lambda i: (0, i, 0)),
            pl.BlockSpec((_BLK, D), lambda i: (i, 0)),
            pl.BlockSpec((_BLK, 1), lambda i: (i, 0)),
            pl.BlockSpec((1, D), lambda i: (0, 0)),
        ],
        out_specs=[
            pl.BlockSpec((_BLK, D), lambda i: (i, 0)),
            pl.BlockSpec((_BLK, D), lambda i: (i, 0)),
        ],
        out_shape=[
            jax.ShapeDtypeStruct((N, D), jnp.float32),
            jax.ShapeDtypeStruct((N, D), jnp.float32),
        ],
    )(rp, u, dinv, b)


def _combine1_kernel(rp_ref, u_ref, dinv_ref, b_ref, z_ref):
    z_ref[...] = dinv_ref[...] * (rp_ref[0] + rp_ref[1] + u_ref[...]) + b_ref[...]


def _combine1(rp, u, dinv, b):
    N, D = u.shape
    return pl.pallas_call(
        _combine1_kernel,
        grid=(_tc_grid(N),),
        in_specs=[
            pl.BlockSpec((_NC, _BLK, D), lambda i: (0, i, 0)),
            pl.BlockSpec((_BLK, D), lambda i: (i, 0)),
            pl.BlockSpec((_BLK, 1), lambda i: (i, 0)),
            pl.BlockSpec((1, D), lambda i: (0, 0)),
        ],
        out_specs=pl.BlockSpec((_BLK, D), lambda i: (i, 0)),
        out_shape=jax.ShapeDtypeStruct((N, D), jnp.float32),
    )(rp, u, dinv, b)


def kernel(x, edge_index, W1, b1, W2, b2, W3, b3, W4, b4):
    N, D = x.shape
    E = edge_index.shape[1]
    ei = edge_index.astype(jnp.int32)
    src, dst = ei[0], ei[1]

    degree = _make_degree(N, E)
    prop = _make_prop(N, E, D)

    degp = degree(dst)
    dinv, u1 = _dinv_and_u(degp, x)

    r1 = prop(u1, src, dst)
    h = _prop_matmul_relu(r1, u1, dinv, W1, b1.reshape(1, -1))
    u2 = _matmul_scale(h, W2, dinv)

    r2 = prop(u2, src, dst)
    z, u3 = _combine2(r2, u2, dinv, b2.reshape(1, -1))

    r3 = prop(u3, src, dst)
    h2 = _prop_matmul_relu(r3, u3, dinv, W3, b3.reshape(1, -1))
    u4 = _matmul_scale(h2, W4, dinv)

    r4 = prop(u4, src, dst)
    recon = _combine1(r4, u4, dinv, b4.reshape(1, -1))
    return (z, recon)
